# baseline (speedup 1.0000x reference)
import jax
import jax.numpy as jnp
from jax import lax
from jax.experimental import pallas as pl
from jax.experimental.pallas import tpu as pltpu
from jax.experimental.pallas import tpu_sc as plsc

BZ = 8
SIZE = 256
NV = 35709
NF = 70789
C = 16
FTOT = BZ * NF
VTOT = BZ * NV
P = BZ * SIZE * SIZE

NC = 2
NS = 16
NW = NC * NS
N = 512
G = N // 128
CHUNKS = P // (NW * N)
L = 16

PCH = 1024
FCT = -(-FTOT // PCH)
VCT = -(-VTOT // PCH)


def _pad_body(faces_hbm, vert_hbm, f16_hbm, xy16_hbm,
              fin_v, vin_v, fout_v, xyout_v, sem):
    cid = lax.axis_index("c")
    sid = lax.axis_index("s")
    wid = sid * NC + cid
    iota = lax.iota(jnp.int32, L)
    zeros16 = jnp.zeros((L,), jnp.int32)
    ones16 = jnp.full((L,), 1, jnp.int32)
    two16 = jnp.full((L,), 2, jnp.int32)

    def face_it(i, carry):
        t = i * NW + wid

        @pl.when(t < FCT)
        def _():
            base = jnp.minimum(t * PCH, FTOT - PCH)
            pltpu.sync_copy(faces_hbm.at[pl.ds(base, PCH)], fin_v)

            def grp(j, c2):
                pvec = iota + j * L
                i0 = plsc.load_gather(fin_v, [pvec, zeros16])
                i1 = plsc.load_gather(fin_v, [pvec, ones16])
                i2 = plsc.load_gather(fin_v, [pvec, two16])
                plsc.store_scatter(fout_v, [pvec, zeros16], i0)
                plsc.store_scatter(fout_v, [pvec, ones16], i1)
                plsc.store_scatter(fout_v, [pvec, two16], i2)
                return c2
            lax.fori_loop(0, PCH // L, grp, 0)
            pltpu.sync_copy(fout_v, f16_hbm.at[pl.ds(base, PCH)])
        return carry

    def vert_it(i, carry):
        t = i * NW + wid

        @pl.when(t < VCT)
        def _():
            base = jnp.minimum(t * PCH, VTOT - PCH)
            pltpu.sync_copy(vert_hbm.at[pl.ds(base, PCH)], vin_v)

            def grp(j, c2):
                pvec = iota + j * L
                x = plsc.load_gather(vin_v, [pvec, zeros16])
                y = plsc.load_gather(vin_v, [pvec, ones16])
                plsc.store_scatter(xyout_v, [pvec, zeros16], x)
                plsc.store_scatter(xyout_v, [pvec, ones16], y)
                return c2
            lax.fori_loop(0, PCH // L, grp, 0)
            pltpu.sync_copy(xyout_v, xy16_hbm.at[pl.ds(base, PCH)])
        return carry

    lax.fori_loop(0, -(-FCT // NW), face_it, 0)
    lax.fori_loop(0, -(-VCT // NW), vert_it, 0)


def _body(attr_hbm, faces_hbm, xy_hbm, p2f_hbm, bary_hbm, out_hbm,
          pix_v, bary_v, fsel_v, frow_v, vidx_v, xy_v, wts_v, arows_v,
          obuf_v, sem):
    cid = lax.axis_index("c")
    sid = lax.axis_index("s")
    wid = sid * NC + cid
    iota = lax.iota(jnp.int32, L)
    zeros16 = jnp.zeros((L,), jnp.int32)
    ones16 = jnp.full((L,), 1, jnp.int32)

    def chunk_body(ci, carry):
        base = (wid * CHUNKS + ci) * N
        pltpu.sync_copy(p2f_hbm.at[pl.ds(base, N)], pix_v)
        pltpu.sync_copy(bary_hbm.at[pl.ds(base, N)], bary_v)

        def fsel_g(g, c2):
            def fsel_l(l, c3):
                f = pix_v[pl.ds(g * 128 + l * L, L)]
                fsel_v[g, pl.ds(l * L, L)] = jnp.maximum(f, 0)
                return c3
            return lax.fori_loop(0, 128 // L, fsel_l, c2)
        lax.fori_loop(0, G, fsel_g, 0)

        hf = [pltpu.async_copy(faces_hbm.at[fsel_v.at[g]], frow_v.at[g], sem)
              for g in range(G)]
        for h in hf:
            h.wait()

        def vidx_g(g, c2):
            def vidx_l(l, c3):
                lanes = iota + l * L
                gv = zeros16 + g
                for k in range(3):
                    ik = plsc.load_gather(
                        frow_v, [gv, lanes, jnp.full((L,), k, jnp.int32)])
                    vidx_v[k, g, pl.ds(l * L, L)] = ik
                return c3
            return lax.fori_loop(0, 128 // L, vidx_l, c2)
        lax.fori_loop(0, G, vidx_g, 0)

        hxy = [pltpu.async_copy(xy_hbm.at[vidx_v.at[k, g]], xy_v.at[k, g], sem)
               for k in range(3) for g in range(G)]
        for h in hxy:
            h.wait()

        ha = [pltpu.async_copy(attr_hbm.at[vidx_v.at[k, g]], arows_v.at[k, g], sem)
              for k in range(3) for g in range(G)]

        def wts_g(g, c2):
            def wts_l(l, c3):
                lanes = iota + l * L
                gv = zeros16 + g
                k0 = zeros16
                k1 = ones16
                k2 = jnp.full((L,), 2, jnp.int32)
                x0 = plsc.load_gather(xy_v, [k0, gv, lanes, zeros16])
                y0 = plsc.load_gather(xy_v, [k0, gv, lanes, ones16])
                x1 = plsc.load_gather(xy_v, [k1, gv, lanes, zeros16])
                y1 = plsc.load_gather(xy_v, [k1, gv, lanes, ones16])
                x2 = plsc.load_gather(xy_v, [k2, gv, lanes, zeros16])
                y2 = plsc.load_gather(xy_v, [k2, gv, lanes, ones16])
                area = (x0 - x1) * (y2 - y1) - (y0 - y1) * (x2 - x1)
                f = pix_v[pl.ds(g * 128 + l * L, L)]
                valid = jnp.logical_and(area > 0.0, f >= 0)
                m = jnp.where(valid, 1.0, 0.0).astype(jnp.float32)
                pvec = lanes + g * 128
                for k in range(3):
                    wk = plsc.load_gather(
                        bary_v, [pvec, jnp.full((L,), k, jnp.int32)]) * m
                    wts_v[k, pl.ds(g * 128 + l * L, L)] = wk
                return c3
            return lax.fori_loop(0, 128 // L, wts_l, c2)
        lax.fori_loop(0, G, wts_g, 0)

        for h in ha:
            h.wait()

        def blend_g(g, c2):
            def blend_l(l, c3):
                w0v = wts_v[0, pl.ds(g * 128 + l * L, L)]
                w1v = wts_v[1, pl.ds(g * 128 + l * L, L)]
                w2v = wts_v[2, pl.ds(g * 128 + l * L, L)]
                for i in range(L):
                    q = l * L + i
                    a0 = arows_v[0, g, q, :]
                    a1 = arows_v[1, g, q, :]
                    a2 = arows_v[2, g, q, :]
                    obuf_v[g * 128 + q, :] = (
                        a0 * w0v[i] + a1 * w1v[i] + a2 * w2v[i])
                return c3
            return lax.fori_loop(0, 128 // L, blend_l, c2)
        lax.fori_loop(0, G, blend_g, 0)

        pltpu.sync_copy(obuf_v, out_hbm.at[pl.ds(base, N)])
        return carry

    lax.fori_loop(0, CHUNKS, chunk_body, 0)


def kernel(vert_attr, vert, faces_packed, pix_to_face, bary_coords):
    attr2d = vert_attr.reshape(VTOT, C)
    vert3 = vert.reshape(VTOT, 3)
    p2f = pix_to_face.reshape(P)
    bary2d = bary_coords.reshape(P, 3)

    mesh = plsc.VectorSubcoreMesh(
        core_axis_name="c", subcore_axis_name="s",
        num_cores=NC, num_subcores=NS)
    cp = pltpu.CompilerParams(
        needs_layout_passes=False, use_tc_tiling_on_sc=False)

    pad = pl.kernel(
        _pad_body,
        out_type=(jax.ShapeDtypeStruct((FTOT, C), jnp.int32),
                  jax.ShapeDtypeStruct((VTOT, C), jnp.float32)),
        mesh=mesh,
        compiler_params=cp,
        scratch_types=[
            pltpu.VMEM((PCH, 3), jnp.int32),
            pltpu.VMEM((PCH, 3), jnp.float32),
            pltpu.VMEM((PCH, C), jnp.int32),
            pltpu.VMEM((PCH, C), jnp.float32),
            pltpu.SemaphoreType.DMA,
        ],
    )
    faces16, xy16 = pad(faces_packed, vert3)

    run = pl.kernel(
        _body,
        out_type=jax.ShapeDtypeStruct((P, C), jnp.float32),
        mesh=mesh,
        compiler_params=cp,
        scratch_types=[
            pltpu.VMEM((N,), jnp.int32),
            pltpu.VMEM((N, 3), jnp.float32),
            pltpu.VMEM((G, 128), jnp.int32),
            pltpu.VMEM((G, 128, C), jnp.int32),
            pltpu.VMEM((3, G, 128), jnp.int32),
            pltpu.VMEM((3, G, 128, C), jnp.float32),
            pltpu.VMEM((3, N), jnp.float32),
            pltpu.VMEM((3, G, 128, C), jnp.float32),
            pltpu.VMEM((N, C), jnp.float32),
            pltpu.SemaphoreType.DMA,
        ],
    )
    flat = run(attr2d, faces16, xy16, p2f, bary2d)
    out = flat.reshape(BZ, SIZE, SIZE, C)
    return jnp.transpose(out, (0, 3, 1, 2))


# speedup vs baseline: 1.0352x; 1.0352x over previous
import jax
import jax.numpy as jnp
from jax import lax
from jax.experimental import pallas as pl
from jax.experimental.pallas import tpu as pltpu
from jax.experimental.pallas import tpu_sc as plsc

BZ = 8
SIZE = 256
NV = 35709
NF = 70789
C = 16
FTOT = BZ * NF
VTOT = BZ * NV
P = BZ * SIZE * SIZE
HW = SIZE * SIZE

NC = 2
NS = 16
NW = NC * NS
N = 512
G = N // 128
CHUNKS = P // (NW * N)
L = 16

PCH = 1024
FCT = -(-FTOT // PCH)
VCT = -(-VTOT // PCH)


def _pad_body(faces_hbm, vert_hbm, f16_hbm, xy16_hbm,
              fin_v, vin_v, fout_v, xyout_v, sem):
    cid = lax.axis_index("c")
    sid = lax.axis_index("s")
    wid = sid * NC + cid
    iota = lax.iota(jnp.int32, L)
    zeros16 = jnp.zeros((L,), jnp.int32)
    ones16 = jnp.full((L,), 1, jnp.int32)
    two16 = jnp.full((L,), 2, jnp.int32)

    def face_it(i, carry):
        t = i * NW + wid

        @pl.when(t < FCT)
        def _():
            base = jnp.minimum(t * PCH, FTOT - PCH)
            pltpu.sync_copy(faces_hbm.at[pl.ds(base, PCH)], fin_v)

            def grp(j, c2):
                pvec = iota + j * L
                i0 = plsc.load_gather(fin_v, [pvec, zeros16])
                i1 = plsc.load_gather(fin_v, [pvec, ones16])
                i2 = plsc.load_gather(fin_v, [pvec, two16])
                plsc.store_scatter(fout_v, [pvec, zeros16], i0)
                plsc.store_scatter(fout_v, [pvec, ones16], i1)
                plsc.store_scatter(fout_v, [pvec, two16], i2)
                return c2
            lax.fori_loop(0, PCH // L, grp, 0)
            pltpu.sync_copy(fout_v, f16_hbm.at[pl.ds(base, PCH)])
        return carry

    def vert_it(i, carry):
        t = i * NW + wid

        @pl.when(t < VCT)
        def _():
            base = jnp.minimum(t * PCH, VTOT - PCH)
            pltpu.sync_copy(vert_hbm.at[pl.ds(base, PCH)], vin_v)

            def grp(j, c2):
                pvec = iota + j * L
                x = plsc.load_gather(vin_v, [pvec, zeros16])
                y = plsc.load_gather(vin_v, [pvec, ones16])
                plsc.store_scatter(xyout_v, [pvec, zeros16], x)
                plsc.store_scatter(xyout_v, [pvec, ones16], y)
                return c2
            lax.fori_loop(0, PCH // L, grp, 0)
            pltpu.sync_copy(xyout_v, xy16_hbm.at[pl.ds(base, PCH)])
        return carry

    lax.fori_loop(0, -(-FCT // NW), face_it, 0)
    lax.fori_loop(0, -(-VCT // NW), vert_it, 0)


def _body(attr_hbm, faces_hbm, xy_hbm, p2f_hbm, bary_hbm, out_hbm,
          pix_v, bary_v, fsel_v, frow_v, vidx_v, xy_v, wts_v, arows_v,
          obuf_v, sem):
    cid = lax.axis_index("c")
    sid = lax.axis_index("s")
    wid = sid * NC + cid
    iota = lax.iota(jnp.int32, L)
    zeros16 = jnp.zeros((L,), jnp.int32)
    ones16 = jnp.full((L,), 1, jnp.int32)

    def chunk_body(ci, carry):
        base = (wid * CHUNKS + ci) * N
        pltpu.sync_copy(p2f_hbm.at[pl.ds(base, N)], pix_v)
        pltpu.sync_copy(bary_hbm.at[pl.ds(base, N)], bary_v)

        def fsel_g(g, c2):
            def fsel_l(l, c3):
                f = pix_v[pl.ds(g * 128 + l * L, L)]
                fsel_v[g, pl.ds(l * L, L)] = jnp.maximum(f, 0)
                return c3
            return lax.fori_loop(0, 128 // L, fsel_l, c2)
        lax.fori_loop(0, G, fsel_g, 0)

        hf = [pltpu.async_copy(faces_hbm.at[fsel_v.at[g]], frow_v.at[g], sem)
              for g in range(G)]
        for h in hf:
            h.wait()

        def vidx_g(g, c2):
            def vidx_l(l, c3):
                lanes = iota + l * L
                gv = zeros16 + g
                for k in range(3):
                    ik = plsc.load_gather(
                        frow_v, [gv, lanes, jnp.full((L,), k, jnp.int32)])
                    vidx_v[k, g, pl.ds(l * L, L)] = ik
                return c3
            return lax.fori_loop(0, 128 // L, vidx_l, c2)
        lax.fori_loop(0, G, vidx_g, 0)

        hxy = [pltpu.async_copy(xy_hbm.at[vidx_v.at[k, g]], xy_v.at[k, g], sem)
               for k in range(3) for g in range(G)]
        for h in hxy:
            h.wait()

        ha = [pltpu.async_copy(attr_hbm.at[vidx_v.at[k, g]], arows_v.at[k, g], sem)
              for k in range(3) for g in range(G)]

        def wts_g(g, c2):
            def wts_l(l, c3):
                lanes = iota + l * L
                gv = zeros16 + g
                k0 = zeros16
                k1 = ones16
                k2 = jnp.full((L,), 2, jnp.int32)
                x0 = plsc.load_gather(xy_v, [k0, gv, lanes, zeros16])
                y0 = plsc.load_gather(xy_v, [k0, gv, lanes, ones16])
                x1 = plsc.load_gather(xy_v, [k1, gv, lanes, zeros16])
                y1 = plsc.load_gather(xy_v, [k1, gv, lanes, ones16])
                x2 = plsc.load_gather(xy_v, [k2, gv, lanes, zeros16])
                y2 = plsc.load_gather(xy_v, [k2, gv, lanes, ones16])
                area = (x0 - x1) * (y2 - y1) - (y0 - y1) * (x2 - x1)
                f = pix_v[pl.ds(g * 128 + l * L, L)]
                valid = jnp.logical_and(area > 0.0, f >= 0)
                m = jnp.where(valid, 1.0, 0.0).astype(jnp.float32)
                pvec = lanes + g * 128
                for k in range(3):
                    wk = plsc.load_gather(
                        bary_v, [pvec, jnp.full((L,), k, jnp.int32)]) * m
                    wts_v[k, pl.ds(g * 128 + l * L, L)] = wk
                return c3
            return lax.fori_loop(0, 128 // L, wts_l, c2)
        lax.fori_loop(0, G, wts_g, 0)

        for h in ha:
            h.wait()

        def blend_g(g, c2):
            def blend_l(l, c3):
                w0v = wts_v[0, pl.ds(g * 128 + l * L, L)]
                w1v = wts_v[1, pl.ds(g * 128 + l * L, L)]
                w2v = wts_v[2, pl.ds(g * 128 + l * L, L)]
                for i in range(L):
                    q = l * L + i
                    a0 = arows_v[0, g, q, :]
                    a1 = arows_v[1, g, q, :]
                    a2 = arows_v[2, g, q, :]
                    ov = a0 * w0v[i] + a1 * w1v[i] + a2 * w2v[i]
                    plsc.store_scatter(
                        obuf_v, [iota, zeros16 + (g * 128 + q)], ov)
                return c3
            return lax.fori_loop(0, 128 // L, blend_l, c2)
        lax.fori_loop(0, G, blend_g, 0)

        b = lax.shift_right_logical(base, 16)
        hw0 = pl.multiple_of(base & (HW - 1), N)
        ho = [pltpu.async_copy(obuf_v.at[c, pl.ds(0, N)],
                               out_hbm.at[b, c, pl.ds(hw0, N)], sem)
              for c in range(C)]
        for h in ho:
            h.wait()
        return carry

    lax.fori_loop(0, CHUNKS, chunk_body, 0)


def kernel(vert_attr, vert, faces_packed, pix_to_face, bary_coords):
    attr2d = vert_attr.reshape(VTOT, C)
    vert3 = vert.reshape(VTOT, 3)
    p2f = pix_to_face.reshape(P)
    bary2d = bary_coords.reshape(P, 3)

    mesh = plsc.VectorSubcoreMesh(
        core_axis_name="c", subcore_axis_name="s",
        num_cores=NC, num_subcores=NS)
    cp = pltpu.CompilerParams(
        needs_layout_passes=False, use_tc_tiling_on_sc=False)

    pad = pl.kernel(
        _pad_body,
        out_type=(jax.ShapeDtypeStruct((FTOT, C), jnp.int32),
                  jax.ShapeDtypeStruct((VTOT, C), jnp.float32)),
        mesh=mesh,
        compiler_params=cp,
        scratch_types=[
            pltpu.VMEM((PCH, 3), jnp.int32),
            pltpu.VMEM((PCH, 3), jnp.float32),
            pltpu.VMEM((PCH, C), jnp.int32),
            pltpu.VMEM((PCH, C), jnp.float32),
            pltpu.SemaphoreType.DMA,
        ],
    )
    faces16, xy16 = pad(faces_packed, vert3)

    run = pl.kernel(
        _body,
        out_type=jax.ShapeDtypeStruct((BZ, C, HW), jnp.float32),
        mesh=mesh,
        compiler_params=cp,
        scratch_types=[
            pltpu.VMEM((N,), jnp.int32),
            pltpu.VMEM((N, 3), jnp.float32),
            pltpu.VMEM((G, 128), jnp.int32),
            pltpu.VMEM((G, 128, C), jnp.int32),
            pltpu.VMEM((3, G, 128), jnp.int32),
            pltpu.VMEM((3, G, 128, C), jnp.float32),
            pltpu.VMEM((3, N), jnp.float32),
            pltpu.VMEM((3, G, 128, C), jnp.float32),
            pltpu.VMEM((C, N + 1), jnp.float32),
            pltpu.SemaphoreType.DMA,
        ],
    )
    out = run(attr2d, faces16, xy16, p2f, bary2d)
    return out.reshape(BZ, C, SIZE, SIZE)
